# baseline (speedup 1.0000x reference)
import functools

import jax
import jax.numpy as jnp
from jax import lax
from jax.experimental import pallas as pl
from jax.experimental.pallas import tpu as pltpu
from jax.experimental.pallas import tpu_sc as plsc

B, T, K = 128, 8192, 16
N = B * T
NC, NS, L = 2, 16, 16
NW = NC * NS
R = 1024
CPW = N // NW // R
NBUF = 4

_mesh = plsc.VectorSubcoreMesh(core_axis_name="c", subcore_axis_name="s")


@functools.partial(
    pl.kernel,
    mesh=_mesh,
    out_type=jax.ShapeDtypeStruct((N,), jnp.float32),
    compiler_params=pltpu.CompilerParams(needs_layout_passes=False),
    scratch_types=[
        pltpu.VMEM((R,), jnp.int32),
        pltpu.VMEM((R,), jnp.int32),
        pltpu.VMEM((R,), jnp.int32),
        pltpu.VMEM((R,), jnp.int32),
        pltpu.VMEM((R,), jnp.float32),
        pltpu.VMEM((R,), jnp.float32),
        pltpu.VMEM((R,), jnp.float32),
        pltpu.VMEM((R,), jnp.float32),
        pltpu.VMEM((R,), jnp.float32),
        pltpu.VMEM((R,), jnp.float32),
        pltpu.VMEM((R,), jnp.float32),
        pltpu.VMEM((R,), jnp.float32),
        pltpu.VMEM((NBUF, 8, 128), jnp.int32),
        pltpu.VMEM((NBUF, R), jnp.float32),
        pltpu.VMEM((NBUF, R), jnp.float32),
        pltpu.SemaphoreType.DMA,
        pltpu.SemaphoreType.DMA,
        pltpu.SemaphoreType.DMA,
        pltpu.SemaphoreType.DMA,
        pltpu.SemaphoreType.DMA,
        pltpu.SemaphoreType.DMA,
        pltpu.SemaphoreType.DMA,
        pltpu.SemaphoreType.DMA,
        pltpu.SemaphoreType.DMA,
        pltpu.SemaphoreType.DMA,
        pltpu.SemaphoreType.DMA,
        pltpu.SemaphoreType.DMA,
    ],
)
def _sc_sample(mus_f, sig_f, z_x, noise_x, out_hbm,
               idx0, idx1, idx2, idx3,
               mug0, mug1, mug2, mug3,
               sgg0, sgg1, sgg2, sgg3,
               z_v, nz_v, out_v,
               lsem0, lsem1, lsem2, lsem3,
               gsem0, gsem1, gsem2, gsem3,
               osem0, osem1, osem2, osem3):
    wid = lax.axis_index("s") * NC + lax.axis_index("c")
    cc0 = wid * CPW

    idx_v = (idx0, idx1, idx2, idx3)
    mu_g = (mug0, mug1, mug2, mug3)
    sg_g = (sgg0, sgg1, sgg2, sgg3)
    lsems = (lsem0, lsem1, lsem2, lsem3)
    gsems = (gsem0, gsem1, gsem2, gsem3)
    osems = (osem0, osem1, osem2, osem3)

    def zn_copies(c, p):
        cc = cc0 + c
        b = cc // 8
        tt0 = (cc % 8) * 8
        zn0 = (b // 8) * 64 + tt0
        zbs = b % 8
        return (
            pltpu.make_async_copy(z_x.at[pl.ds(zn0, 8), zbs], z_v.at[p], lsems[p]),
            pltpu.make_async_copy(noise_x.at[pl.ds(cc * R, R)], nz_v.at[p], lsems[p]),
        )

    def gather_copies(p):
        return (
            pltpu.make_async_copy(mus_f.at[idx_v[p]], mu_g[p], gsems[p]),
            pltpu.make_async_copy(sig_f.at[idx_v[p]], sg_g[p], gsems[p]),
        )

    def store_copy(c, p):
        return pltpu.make_async_copy(
            out_v.at[p], out_hbm.at[pl.ds((cc0 + c) * R, R)], osems[p])

    def idx_compute(c, p):
        cc = cc0 + c
        b = cc // 8
        sb = b * 131072 + (cc % 8) * 8192

        def vec_body(i, carry):
            tv = lax.iota(jnp.int32, L) + i * L
            zv = z_v[p, i // 8, pl.ds((i % 8) * L, L)]
            idx_v[p][pl.ds(i * L, L)] = (
                sb + ((zv >> 3) << 16) + ((tv >> 7) << 10)
                + ((zv & 7) << 7) + (tv & 127))
            return carry

        lax.fori_loop(0, R // L, vec_body, 0, unroll=8)

    def out_compute(p):
        def vec_body(i, carry):
            s = pl.ds(i * L, L)
            out_v[p, s] = mu_g[p][s] + sg_g[p][s] * nz_v[p, s]
            return carry

        lax.fori_loop(0, R // L, vec_body, 0, unroll=8)

    def step(c, p):
        pm1 = (p - 1) % NBUF
        for d in zn_copies(c, p):
            d.wait()
        idx_compute(c, p)
        for d in gather_copies(p):
            d.start()
        nxt = jnp.minimum(c + 1, CPW - 1)
        for d in zn_copies(nxt, (p + 1) % NBUF):
            d.start()

        @pl.when(c > 0)
        def _():
            for d in gather_copies(pm1):
                d.wait()

            @pl.when(c - 1 >= NBUF)
            def _():
                store_copy(c - 1 - NBUF, pm1).wait()

            out_compute(pm1)
            store_copy(c - 1, pm1).start()

    for d in zn_copies(0, 0):
        d.start()

    def quad_body(g, carry):
        for j in range(NBUF):
            step(NBUF * g + j, j)
        return carry

    lax.fori_loop(0, CPW // NBUF, quad_body, 0)

    last = CPW - 1
    pl_ = last % NBUF
    for d in zn_copies(last, (last + 1) % NBUF):
        d.wait()
    for d in gather_copies(pl_):
        d.wait()
    store_copy(last - NBUF, pl_).wait()
    out_compute(pl_)
    store_copy(last, pl_).start()
    for m in range(NBUF):
        store_copy(last - m, (last - m) % NBUF).wait()


def kernel(mus, sigmas, z, noise):
    mus_f = (mus.reshape(B, 64, 128, 2, 8)
             .transpose(0, 3, 1, 4, 2)
             .reshape(-1))
    sig_f = (sigmas.reshape(B, 64, 128, 2, 8)
             .transpose(0, 3, 1, 4, 2)
             .reshape(-1))
    z_x = (z.astype(jnp.int32)
           .reshape(16, 8, 64, 128)
           .transpose(0, 2, 1, 3)
           .reshape(1024, 8, 128))
    noise_x = noise.reshape(-1)
    out = _sc_sample(mus_f, sig_f, z_x, noise_x)
    return out.reshape(B, T, 1)
